# baseline (speedup 1.0000x reference)
import functools

import jax
import jax.numpy as jnp
from jax import lax
from jax.experimental import pallas as pl
from jax.experimental.pallas import tpu as pltpu
from jax.experimental.pallas import tpu_sc as plsc

_K = 8


def _topk_body(gp_ref, mv_ref, m2_ref, idx_ref, w_ref):
    gp = gp_ref[...]
    gx, gy, gz = gp[:, 0:1], gp[:, 1:2], gp[:, 2:3]
    a2 = gx * gx + gy * gy + gz * gz
    dot = jax.lax.dot_general(
        gp.astype(jnp.bfloat16), mv_ref[...].astype(jnp.bfloat16),
        (((1,), (0,)), ((), ())),
        preferred_element_type=jnp.float32)
    sq = a2 + m2_ref[...] - 2.0 * dot
    d = jnp.sqrt(jnp.maximum(sq, 0.0) + 1e-12)

    br, np_ = d.shape
    col = jax.lax.broadcasted_iota(jnp.int32, (br, np_), 1)
    sumexp = jnp.zeros((br, 1), jnp.float32)
    d0 = jnp.zeros((br, 1), jnp.float32)
    evals = []
    for t in range(_K):
        dmin = jnp.min(d, axis=1, keepdims=True)
        idxc = jnp.where(d == dmin, col, jnp.int32(2 ** 30))
        amin = jnp.min(idxc, axis=1)
        if t == 0:
            d0 = dmin
        e = jnp.exp(d0 - dmin)
        sumexp = sumexp + e
        evals.append(e)
        idx_ref[:, t] = amin
        if t < _K - 1:
            mask = col == amin[:, None]
            d = jnp.where(mask, jnp.float32(3.0e38), d)
    inv = 1.0 / sumexp
    w_ref[...] = jnp.concatenate(evals, axis=1) * inv


def _proj_body(x_ref, wt_ref, bb_ref, out_ref):
    out_ref[...] = jax.lax.dot_general(
        x_ref[...].astype(jnp.bfloat16), wt_ref[...].astype(jnp.bfloat16),
        (((1,), (0,)), ((), ())),
        preferred_element_type=jnp.float32) + bb_ref[...]


def _round_up(x, m):
    return (x + m - 1) // m * m


def _make_sc_combine(total, n_rows, C, pairs_per_w, P, NC):
    mesh = plsc.VectorSubcoreMesh(core_axis_name="c", subcore_axis_name="s")

    @functools.partial(
        pl.kernel, mesh=mesh,
        out_type=jax.ShapeDtypeStruct((total, C), jnp.float32),
        scratch_types=[
            pltpu.VMEM((P * _K,), jnp.int32),
            pltpu.VMEM((P * _K, 128), jnp.float32),
            pltpu.VMEM((P, _K, 16), jnp.float32),
            pltpu.VMEM((P, C), jnp.float32),
            pltpu.SemaphoreType.DMA,
        ],
    )
    def sc_combine(mesh_hbm, idx_hbm, w_hbm, out_hbm, idx_v, rows_v, w_v,
                   out_v, sem):
        wid = lax.axis_index("s") * NC + lax.axis_index("c")
        base = wid * pairs_per_w

        def chunk_body(ci, carry):
            start = base + ci * P
            pltpu.sync_copy(idx_hbm.at[pl.ds(start * _K, P * _K)], idx_v)
            pltpu.async_copy(mesh_hbm.at[idx_v], rows_v, sem).wait()
            pltpu.sync_copy(w_hbm.at[pl.ds(start, P)], w_v)

            def pt_body(p, c2):
                accs = [jnp.zeros((16,), jnp.float32) for _ in range(C // 16)]
                for k in range(_K):
                    wk = w_v[p, k, :]
                    for c4 in range(C // 16):
                        accs[c4] = accs[c4] + wk * rows_v[p * _K + k,
                                                          pl.ds(c4 * 16, 16)]
                for c4 in range(C // 16):
                    out_v[p, pl.ds(c4 * 16, 16)] = accs[c4]
                return c2

            lax.fori_loop(0, P, pt_body, 0)
            pltpu.sync_copy(out_v, out_hbm.at[pl.ds(start, P)])
            return carry

        lax.fori_loop(0, pairs_per_w // P, chunk_body, 0)

    return sc_combine


@jax.jit
def kernel(mesh_output, mesh_vertices, lat, lon, W, b):
    B, n_mesh, C = mesh_output.shape
    n_lat, n_lon = lat.shape[0], lon.shape[0]
    G = n_lat * n_lon
    BR = 512
    GP = _round_up(G, BR)
    NP = _round_up(n_mesh, 128)

    cl, sl = jnp.cos(lat), jnp.sin(lat)
    clo, slo = jnp.cos(lon), jnp.sin(lon)
    x = cl[:, None] * clo[None, :]
    y = cl[:, None] * slo[None, :]
    z = jnp.broadcast_to(sl[:, None], (n_lat, n_lon))
    gp = jnp.stack([x, y, z], axis=-1).reshape(G, 3)
    gp = jnp.pad(gp, ((0, GP - G), (0, 8 - 3)))

    mv = jnp.pad(mesh_vertices, ((0, NP - n_mesh), (0, 8 - 3))).T
    m2 = jnp.sum(mesh_vertices * mesh_vertices, axis=1)
    m2 = jnp.pad(m2, (0, NP - n_mesh), constant_values=1.0e6)[None, :]

    knn_idx, w = pl.pallas_call(
        _topk_body,
        grid=(GP // BR,),
        in_specs=[
            pl.BlockSpec((BR, 8), lambda i: (i, 0)),
            pl.BlockSpec((8, NP), lambda i: (0, 0)),
            pl.BlockSpec((1, NP), lambda i: (0, 0)),
        ],
        out_specs=[
            pl.BlockSpec((BR, _K), lambda i: (i, 0)),
            pl.BlockSpec((BR, _K), lambda i: (i, 0)),
        ],
        out_shape=[
            jax.ShapeDtypeStruct((GP, _K), jnp.int32),
            jax.ShapeDtypeStruct((GP, _K), jnp.float32),
        ],
    )(gp, mv, m2)

    total = B * GP
    mesh_flat = jnp.pad(mesh_output,
                        ((0, 0), (0, NP - n_mesh),
                         (0, 128 - C))).reshape(B * NP, 128)
    idx_flat = (knn_idx[None, :, :]
                + (jnp.arange(B, dtype=jnp.int32) * NP)[:, None, None]
                ).reshape(total * _K)
    w_rep = jnp.broadcast_to(w[None, :, :, None],
                             (B, GP, _K, 16)).reshape(total, _K, 16)

    info = plsc.get_sparse_core_info()
    NW = info.num_cores * info.num_subcores
    pairs_per_w = total // NW
    P = 16
    comb = _make_sc_combine(total, B * NP, C, pairs_per_w, P,
                            info.num_cores)(mesh_flat, idx_flat, w_rep)

    out = pl.pallas_call(
        _proj_body,
        grid=(total // 512,),
        in_specs=[
            pl.BlockSpec((512, C), lambda i: (i, 0)),
            pl.BlockSpec((C, C), lambda i: (0, 0)),
            pl.BlockSpec((1, C), lambda i: (0, 0)),
        ],
        out_specs=pl.BlockSpec((512, C), lambda i: (i, 0)),
        out_shape=jax.ShapeDtypeStruct((total, C), jnp.float32),
    )(comb, W.T, b[None, :])

    out = out.reshape(B, GP, C)[:, :G].reshape(B, n_lat, n_lon, C)
    return jnp.transpose(out, (0, 3, 1, 2))


# speedup vs baseline: 1.1413x; 1.1413x over previous
import functools

import jax
import jax.numpy as jnp
from jax import lax
from jax.experimental import pallas as pl
from jax.experimental.pallas import tpu as pltpu
from jax.experimental.pallas import tpu_sc as plsc

_K = 8


def _topk_body(gp_ref, mv_ref, m2_ref, idx_ref, w_ref):
    gp = gp_ref[...]
    gx, gy, gz = gp[:, 0:1], gp[:, 1:2], gp[:, 2:3]
    a2 = gx * gx + gy * gy + gz * gz
    dot = jax.lax.dot_general(
        gp.astype(jnp.bfloat16), mv_ref[...].astype(jnp.bfloat16),
        (((1,), (0,)), ((), ())),
        preferred_element_type=jnp.float32)
    sq = a2 + m2_ref[...] - 2.0 * dot
    d = jnp.sqrt(jnp.maximum(sq, 0.0) + 1e-12)

    br, np_ = d.shape
    col = jax.lax.broadcasted_iota(jnp.int32, (br, np_), 1)
    sumexp = jnp.zeros((br, 1), jnp.float32)
    d0 = jnp.zeros((br, 1), jnp.float32)
    evals = []
    for t in range(_K):
        dmin = jnp.min(d, axis=1, keepdims=True)
        idxc = jnp.where(d == dmin, col, jnp.int32(2 ** 30))
        amin = jnp.min(idxc, axis=1)
        if t == 0:
            d0 = dmin
        e = jnp.exp(d0 - dmin)
        sumexp = sumexp + e
        evals.append(e)
        idx_ref[:, t] = amin
        if t < _K - 1:
            mask = col == amin[:, None]
            d = jnp.where(mask, jnp.float32(3.0e38), d)
    inv = 1.0 / sumexp
    w_ref[...] = jnp.concatenate(evals, axis=1) * inv


def _proj_body(x_ref, wt_ref, bb_ref, out_ref):
    out_ref[...] = jax.lax.dot_general(
        x_ref[...].astype(jnp.bfloat16), wt_ref[...].astype(jnp.bfloat16),
        (((1,), (0,)), ((), ())),
        preferred_element_type=jnp.float32) + bb_ref[...]


def _round_up(x, m):
    return (x + m - 1) // m * m


def _make_sc_combine(total, n_rows, C, pairs_per_w, P, NC):
    mesh = plsc.VectorSubcoreMesh(core_axis_name="c", subcore_axis_name="s")

    @functools.partial(
        pl.kernel, mesh=mesh,
        out_type=jax.ShapeDtypeStruct((total, C), jnp.float32),
        scratch_types=[
            pltpu.VMEM((P * _K,), jnp.int32),
            pltpu.VMEM((P * _K, 128), jnp.float32),
            pltpu.VMEM((P, _K, 16), jnp.float32),
            pltpu.VMEM((P, C), jnp.float32),
            pltpu.SemaphoreType.DMA,
        ],
    )
    def sc_combine(mesh_hbm, idx_hbm, w_hbm, out_hbm, idx_v, rows_v, w_v,
                   out_v, sem):
        wid = lax.axis_index("s") * NC + lax.axis_index("c")
        base = wid * pairs_per_w

        def chunk_body(ci, carry):
            start = base + ci * P
            pltpu.sync_copy(idx_hbm.at[pl.ds(start * _K, P * _K)], idx_v)
            pltpu.async_copy(mesh_hbm.at[idx_v], rows_v, sem).wait()
            pltpu.sync_copy(w_hbm.at[pl.ds(start, P)], w_v)

            def pt_body(p, c2):
                accs = [jnp.zeros((16,), jnp.float32) for _ in range(C // 16)]
                for k in range(_K):
                    wk = w_v[p, k, :]
                    for c4 in range(C // 16):
                        accs[c4] = accs[c4] + wk * rows_v[p * _K + k,
                                                          pl.ds(c4 * 16, 16)]
                for c4 in range(C // 16):
                    out_v[p, pl.ds(c4 * 16, 16)] = accs[c4]
                return c2

            lax.fori_loop(0, P, pt_body, 0)
            pltpu.sync_copy(out_v, out_hbm.at[pl.ds(start, P)])
            return carry

        lax.fori_loop(0, pairs_per_w // P, chunk_body, 0)

    return sc_combine


@jax.jit
def kernel(mesh_output, mesh_vertices, lat, lon, W, b):
    B, n_mesh, C = mesh_output.shape
    n_lat, n_lon = lat.shape[0], lon.shape[0]
    G = n_lat * n_lon
    BR = 256
    GP = _round_up(G, BR)
    NP = _round_up(n_mesh, 128)

    cl, sl = jnp.cos(lat), jnp.sin(lat)
    clo, slo = jnp.cos(lon), jnp.sin(lon)
    x = cl[:, None] * clo[None, :]
    y = cl[:, None] * slo[None, :]
    z = jnp.broadcast_to(sl[:, None], (n_lat, n_lon))
    gp = jnp.stack([x, y, z], axis=-1).reshape(G, 3)
    gp = jnp.pad(gp, ((0, GP - G), (0, 8 - 3)))

    mv = jnp.pad(mesh_vertices, ((0, NP - n_mesh), (0, 8 - 3))).T
    m2 = jnp.sum(mesh_vertices * mesh_vertices, axis=1)
    m2 = jnp.pad(m2, (0, NP - n_mesh), constant_values=1.0e6)[None, :]

    knn_idx, w = pl.pallas_call(
        _topk_body,
        grid=(GP // BR,),
        in_specs=[
            pl.BlockSpec((BR, 8), lambda i: (i, 0)),
            pl.BlockSpec((8, NP), lambda i: (0, 0)),
            pl.BlockSpec((1, NP), lambda i: (0, 0)),
        ],
        out_specs=[
            pl.BlockSpec((BR, _K), lambda i: (i, 0)),
            pl.BlockSpec((BR, _K), lambda i: (i, 0)),
        ],
        out_shape=[
            jax.ShapeDtypeStruct((GP, _K), jnp.int32),
            jax.ShapeDtypeStruct((GP, _K), jnp.float32),
        ],
    )(gp, mv, m2)

    total = B * GP
    mesh_flat = jnp.pad(mesh_output,
                        ((0, 0), (0, NP - n_mesh),
                         (0, 128 - C))).reshape(B * NP, 128)
    idx_flat = (knn_idx[None, :, :]
                + (jnp.arange(B, dtype=jnp.int32) * NP)[:, None, None]
                ).reshape(total * _K)
    w_rep = jnp.broadcast_to(w[None, :, :, None],
                             (B, GP, _K, 16)).reshape(total, _K, 16)

    info = plsc.get_sparse_core_info()
    NW = info.num_cores * info.num_subcores
    pairs_per_w = total // NW
    P = 16
    comb = _make_sc_combine(total, B * NP, C, pairs_per_w, P,
                            info.num_cores)(mesh_flat, idx_flat, w_rep)

    out = pl.pallas_call(
        _proj_body,
        grid=(total // 512,),
        in_specs=[
            pl.BlockSpec((512, C), lambda i: (i, 0)),
            pl.BlockSpec((C, C), lambda i: (0, 0)),
            pl.BlockSpec((1, C), lambda i: (0, 0)),
        ],
        out_specs=pl.BlockSpec((512, C), lambda i: (i, 0)),
        out_shape=jax.ShapeDtypeStruct((total, C), jnp.float32),
    )(comb, W.T, b[None, :])

    out = out.reshape(B, GP, C)[:, :G].reshape(B, n_lat, n_lon, C)
    return jnp.transpose(out, (0, 3, 1, 2))


# speedup vs baseline: 1.1769x; 1.0312x over previous
import functools

import jax
import jax.numpy as jnp
from jax import lax
from jax.experimental import pallas as pl
from jax.experimental.pallas import tpu as pltpu
from jax.experimental.pallas import tpu_sc as plsc

_K = 8


def _topk_body(gp_ref, mv_ref, m2_ref, idx_ref, w_ref):
    gp = gp_ref[...]
    gx, gy, gz = gp[:, 0:1], gp[:, 1:2], gp[:, 2:3]
    a2 = gx * gx + gy * gy + gz * gz
    dot = jax.lax.dot_general(
        gp.astype(jnp.bfloat16), mv_ref[...].astype(jnp.bfloat16),
        (((1,), (0,)), ((), ())),
        preferred_element_type=jnp.float32)
    sq = a2 + m2_ref[...] - 2.0 * dot
    d = jnp.sqrt(jnp.maximum(sq, 0.0) + 1e-12)

    br, np_ = d.shape
    col = jax.lax.broadcasted_iota(jnp.int32, (br, np_), 1)
    sumexp = jnp.zeros((br, 1), jnp.float32)
    d0 = jnp.zeros((br, 1), jnp.float32)
    evals = []
    for t in range(_K):
        dmin = jnp.min(d, axis=1, keepdims=True)
        idxc = jnp.where(d == dmin, col, jnp.int32(2 ** 30))
        amin = jnp.min(idxc, axis=1)
        if t == 0:
            d0 = dmin
        e = jnp.exp(d0 - dmin)
        sumexp = sumexp + e
        evals.append(e)
        idx_ref[:, t] = amin
        if t < _K - 1:
            mask = col == amin[:, None]
            d = jnp.where(mask, jnp.float32(3.0e38), d)
    inv = 1.0 / sumexp
    w_ref[...] = jnp.concatenate(evals, axis=1) * inv


def _proj_body(x_ref, wt_ref, bb_ref, out_ref):
    out_ref[...] = jax.lax.dot_general(
        x_ref[...].astype(jnp.bfloat16), wt_ref[...].astype(jnp.bfloat16),
        (((1,), (0,)), ((), ())),
        preferred_element_type=jnp.float32) + bb_ref[...]


def _round_up(x, m):
    return (x + m - 1) // m * m


def _make_sc_combine(total, n_rows, C, pairs_per_w, P, NC):
    mesh = plsc.VectorSubcoreMesh(core_axis_name="c", subcore_axis_name="s")

    @functools.partial(
        pl.kernel, mesh=mesh,
        out_type=jax.ShapeDtypeStruct((total, C), jnp.float32),
        scratch_types=[
            pltpu.VMEM((P * _K,), jnp.int32),
            pltpu.VMEM((P * _K, 128), jnp.float32),
            pltpu.VMEM((P, _K, 16), jnp.float32),
            pltpu.VMEM((P, C), jnp.float32),
            pltpu.SemaphoreType.DMA,
        ],
    )
    def sc_combine(mesh_hbm, idx_hbm, w_hbm, out_hbm, idx_v, rows_v, w_v,
                   out_v, sem):
        wid = lax.axis_index("s") * NC + lax.axis_index("c")
        base = wid * pairs_per_w

        def chunk_body(ci, carry):
            start = base + ci * P
            pltpu.sync_copy(idx_hbm.at[pl.ds(start * _K, P * _K)], idx_v)
            gather = pltpu.async_copy(mesh_hbm.at[idx_v], rows_v, sem)
            pltpu.sync_copy(w_hbm.at[pl.ds(start, P)], w_v)
            gather.wait()

            def pt_body(p, c2):
                accs = [jnp.zeros((16,), jnp.float32) for _ in range(C // 16)]
                for k in range(_K):
                    wk = w_v[p, k, :]
                    for c4 in range(C // 16):
                        accs[c4] = accs[c4] + wk * rows_v[p * _K + k,
                                                          pl.ds(c4 * 16, 16)]
                for c4 in range(C // 16):
                    out_v[p, pl.ds(c4 * 16, 16)] = accs[c4]
                return c2

            lax.fori_loop(0, P, pt_body, 0)
            pltpu.sync_copy(out_v, out_hbm.at[pl.ds(start, P)])
            return carry

        lax.fori_loop(0, pairs_per_w // P, chunk_body, 0)

    return sc_combine


@jax.jit
def kernel(mesh_output, mesh_vertices, lat, lon, W, b):
    B, n_mesh, C = mesh_output.shape
    n_lat, n_lon = lat.shape[0], lon.shape[0]
    G = n_lat * n_lon
    BR = 256
    GP = _round_up(G, BR)
    NP = _round_up(n_mesh, 128)

    cl, sl = jnp.cos(lat), jnp.sin(lat)
    clo, slo = jnp.cos(lon), jnp.sin(lon)
    x = cl[:, None] * clo[None, :]
    y = cl[:, None] * slo[None, :]
    z = jnp.broadcast_to(sl[:, None], (n_lat, n_lon))
    gp = jnp.stack([x, y, z], axis=-1).reshape(G, 3)
    gp = jnp.pad(gp, ((0, GP - G), (0, 8 - 3)))

    mv = jnp.pad(mesh_vertices, ((0, NP - n_mesh), (0, 8 - 3))).T
    m2 = jnp.sum(mesh_vertices * mesh_vertices, axis=1)
    m2 = jnp.pad(m2, (0, NP - n_mesh), constant_values=1.0e6)[None, :]

    knn_idx, w = pl.pallas_call(
        _topk_body,
        grid=(GP // BR,),
        in_specs=[
            pl.BlockSpec((BR, 8), lambda i: (i, 0)),
            pl.BlockSpec((8, NP), lambda i: (0, 0)),
            pl.BlockSpec((1, NP), lambda i: (0, 0)),
        ],
        out_specs=[
            pl.BlockSpec((BR, _K), lambda i: (i, 0)),
            pl.BlockSpec((BR, _K), lambda i: (i, 0)),
        ],
        out_shape=[
            jax.ShapeDtypeStruct((GP, _K), jnp.int32),
            jax.ShapeDtypeStruct((GP, _K), jnp.float32),
        ],
    )(gp, mv, m2)

    total = B * GP
    mesh_flat = jnp.pad(mesh_output,
                        ((0, 0), (0, NP - n_mesh),
                         (0, 128 - C))).reshape(B * NP, 128)
    idx_flat = (knn_idx[None, :, :]
                + (jnp.arange(B, dtype=jnp.int32) * NP)[:, None, None]
                ).reshape(total * _K)
    w_rep = jnp.broadcast_to(w[None, :, :, None],
                             (B, GP, _K, 16)).reshape(total, _K, 16)

    info = plsc.get_sparse_core_info()
    NW = info.num_cores * info.num_subcores
    pairs_per_w = total // NW
    P = 16
    comb = _make_sc_combine(total, B * NP, C, pairs_per_w, P,
                            info.num_cores)(mesh_flat, idx_flat, w_rep)

    out = pl.pallas_call(
        _proj_body,
        grid=(total // 512,),
        in_specs=[
            pl.BlockSpec((512, C), lambda i: (i, 0)),
            pl.BlockSpec((C, C), lambda i: (0, 0)),
            pl.BlockSpec((1, C), lambda i: (0, 0)),
        ],
        out_specs=pl.BlockSpec((512, C), lambda i: (i, 0)),
        out_shape=jax.ShapeDtypeStruct((total, C), jnp.float32),
    )(comb, W.T, b[None, :])

    out = out.reshape(B, GP, C)[:, :G].reshape(B, n_lat, n_lon, C)
    return jnp.transpose(out, (0, 3, 1, 2))
